# baseline (speedup 1.0000x reference)
import functools

import jax
import jax.numpy as jnp
from jax import lax
from jax.experimental import pallas as pl
from jax.experimental.pallas import tpu as pltpu
from jax.experimental.pallas import tpu_sc as plsc

E = 8
K = 2
D = 768
EP = 128
T = 2048
A = K * T
BM3 = 256
NC = 2
NS = 16
NW = NC * NS
TPW = T // NW


def _cumsum0(x):
    c = x
    s = 1
    n = x.shape[0]
    while s < n:
        shifted = jnp.concatenate(
            [jnp.zeros((s, x.shape[1]), c.dtype), c[:n - s]], axis=0)
        c = c + shifted
        s *= 2
    return c


def _routing_body(x_ref, wg_ref, pos0_ref, pos1_ref, w0b_ref, w1b_ref,
                  offs_ref):
    logits = jnp.dot(x_ref[...], wg_ref[...],
                     preferred_element_type=jnp.float32)
    lane = lax.broadcasted_iota(jnp.int32, logits.shape, 1)
    logits = jnp.where(lane < E, logits, -1e30)

    v0 = jnp.max(logits, axis=1, keepdims=True)
    a0 = jnp.min(jnp.where(logits == v0, lane, EP), axis=1, keepdims=True)
    logits2 = jnp.where(lane == a0, -1e30, logits)
    v1 = jnp.max(logits2, axis=1, keepdims=True)
    a1 = jnp.min(jnp.where(logits2 == v1, lane, EP), axis=1, keepdims=True)

    w0 = 1.0 / (1.0 + jnp.exp(v1 - v0))
    w1 = 1.0 - w0

    oh0 = (lane == a0).astype(jnp.int32)
    oh1 = (lane == a1).astype(jnp.int32)
    c0 = _cumsum0(oh0)
    c1 = _cumsum0(oh1)
    cnt0 = c0[T - 1:T, :]
    cnt1 = c1[T - 1:T, :]
    cnt = cnt0 + cnt1

    offs = jnp.concatenate(
        [jnp.zeros((1, 1), jnp.int32), cnt[:, :EP - 1]], axis=1)
    s = 1
    while s < EP:
        offs = offs + jnp.concatenate(
            [jnp.zeros((1, s), jnp.int32), offs[:, :EP - s]], axis=1)
        s *= 2

    pos0 = jnp.sum(oh0 * (offs + c0 - 1), axis=1, keepdims=True)
    pos1 = jnp.sum(oh1 * (offs + cnt0 + c1 - 1), axis=1, keepdims=True)

    pos0_ref[...] = pos0
    pos1_ref[...] = pos1
    w0b_ref[...] = jnp.broadcast_to(w0, (T, 16))
    w1b_ref[...] = jnp.broadcast_to(w1, (T, 16))
    offs_ref[...] = offs


def _routing(xs, wg_pad):
    return pl.pallas_call(
        _routing_body,
        grid=(1,),
        in_specs=[
            pl.BlockSpec((T, D), lambda i: (0, 0)),
            pl.BlockSpec((D, EP), lambda i: (0, 0)),
        ],
        out_specs=[
            pl.BlockSpec((T, 1), lambda i: (0, 0)),
            pl.BlockSpec((T, 1), lambda i: (0, 0)),
            pl.BlockSpec((T, 16), lambda i: (0, 0)),
            pl.BlockSpec((T, 16), lambda i: (0, 0)),
            pl.BlockSpec((1, EP), lambda i: (0, 0)),
        ],
        out_shape=[
            jax.ShapeDtypeStruct((T, 1), jnp.int32),
            jax.ShapeDtypeStruct((T, 1), jnp.int32),
            jax.ShapeDtypeStruct((T, 16), jnp.float32),
            jax.ShapeDtypeStruct((T, 16), jnp.float32),
            jax.ShapeDtypeStruct((1, EP), jnp.int32),
        ],
    )(xs, wg_pad)


def _dispatch_body(xs_hbm, pos0_hbm, pos1_hbm, xg_hbm,
                   idx0, idx1, rows, s0, s1):
    wid = lax.axis_index("s") * NC + lax.axis_index("c")
    base = wid * TPW
    pltpu.sync_copy(pos0_hbm.at[pl.ds(base, TPW)], idx0)
    pltpu.sync_copy(pos1_hbm.at[pl.ds(base, TPW)], idx1)
    pltpu.sync_copy(xs_hbm.at[pl.ds(base, TPW)], rows)
    cp0 = pltpu.async_copy(rows, xg_hbm.at[idx0], s0)
    cp1 = pltpu.async_copy(rows, xg_hbm.at[idx1], s1)
    cp0.wait()
    cp1.wait()


@functools.lru_cache(maxsize=None)
def _make_dispatch():
    return pl.kernel(
        _dispatch_body,
        out_type=jax.ShapeDtypeStruct((A, D), jnp.float32),
        mesh=plsc.VectorSubcoreMesh(core_axis_name="c",
                                    subcore_axis_name="s"),
        scratch_types=[
            pltpu.VMEM((TPW,), jnp.int32),
            pltpu.VMEM((TPW,), jnp.int32),
            pltpu.VMEM((TPW, D), jnp.float32),
            pltpu.SemaphoreType.DMA,
            pltpu.SemaphoreType.DMA,
        ],
    )


def _gmm_body(offs_ref, xg_ref, we_ref, be_ref, y_ref):
    g = pl.program_id(0)
    r0 = g * BM3
    riota = lax.broadcasted_iota(jnp.int32, (BM3, 1), 0) + r0
    y_ref[...] = jnp.zeros((BM3, D), jnp.float32)
    for e in range(E):
        s = offs_ref[e]
        t = offs_ref[e + 1]

        @pl.when((t > r0) & (s < r0 + BM3))
        def _expert(e=e, s=s, t=t):
            m = (riota >= s) & (riota < t)
            y_ref[...] += jnp.where(m, 1.0, 0.0) * (
                jnp.dot(xg_ref[...], we_ref[e],
                        preferred_element_type=jnp.float32)
                + be_ref[e][None, :])


def _grouped_matmul(offs_flat, Xg, We, be):
    grid_spec = pltpu.PrefetchScalarGridSpec(
        num_scalar_prefetch=1,
        grid=(A // BM3,),
        in_specs=[
            pl.BlockSpec((BM3, D), lambda g, offs: (g, 0)),
            pl.BlockSpec((E, D, D), lambda g, offs: (0, 0, 0)),
            pl.BlockSpec((E, D), lambda g, offs: (0, 0)),
        ],
        out_specs=pl.BlockSpec((BM3, D), lambda g, offs: (g, 0)),
    )
    return pl.pallas_call(
        _gmm_body,
        grid_spec=grid_spec,
        out_shape=jax.ShapeDtypeStruct((A, D), jnp.float32),
    )(offs_flat, Xg, We, be)


def _combine_body(y_hbm, pos0_hbm, pos1_hbm, w0b_hbm, w1b_hbm, out_hbm,
                  idx0, idx1, rowsA, rowsB, wv0, wv1, s0, s1):
    wid = lax.axis_index("s") * NC + lax.axis_index("c")
    base = wid * TPW
    pltpu.sync_copy(pos0_hbm.at[pl.ds(base, TPW)], idx0)
    pltpu.sync_copy(pos1_hbm.at[pl.ds(base, TPW)], idx1)
    pltpu.sync_copy(w0b_hbm.at[pl.ds(base, TPW)], wv0)
    pltpu.sync_copy(w1b_hbm.at[pl.ds(base, TPW)], wv1)
    cpA = pltpu.async_copy(y_hbm.at[idx0], rowsA, s0)
    cpB = pltpu.async_copy(y_hbm.at[idx1], rowsB, s1)
    cpA.wait()
    cpB.wait()

    def row_body(r, _):
        w0v = wv0[r, :]
        w1v = wv1[r, :]

        def col_body(c, _):
            col = c * 16
            rowsA[r, pl.ds(col, 16)] = (
                rowsA[r, pl.ds(col, 16)] * w0v
                + rowsB[r, pl.ds(col, 16)] * w1v)
            return 0

        lax.fori_loop(0, D // 16, col_body, 0)
        return 0

    lax.fori_loop(0, TPW, row_body, 0)
    pltpu.sync_copy(rowsA, out_hbm.at[pl.ds(base, TPW)])


@functools.lru_cache(maxsize=None)
def _make_combine():
    return pl.kernel(
        _combine_body,
        out_type=jax.ShapeDtypeStruct((T, D), jnp.float32),
        mesh=plsc.VectorSubcoreMesh(core_axis_name="c",
                                    subcore_axis_name="s"),
        scratch_types=[
            pltpu.VMEM((TPW,), jnp.int32),
            pltpu.VMEM((TPW,), jnp.int32),
            pltpu.VMEM((TPW, D), jnp.float32),
            pltpu.VMEM((TPW, D), jnp.float32),
            pltpu.VMEM((TPW, 16), jnp.float32),
            pltpu.VMEM((TPW, 16), jnp.float32),
            pltpu.SemaphoreType.DMA,
            pltpu.SemaphoreType.DMA,
        ],
    )


@jax.jit
def _moe(xs, wg_pad, We, be):
    pos0, pos1, w0b, w1b, offs = _routing(xs, wg_pad)
    pos0v = pos0.reshape(T)
    pos1v = pos1.reshape(T)
    Xg = _make_dispatch()(xs, pos0v, pos1v)
    Y = _grouped_matmul(offs.reshape(EP), Xg, We, be)
    return _make_combine()(Y, pos0v, pos1v, w0b, w1b)


def kernel(x, Wg, We, be):
    xs = x.reshape(-1, x.shape[-1])
    wg_pad = jnp.pad(Wg, ((0, 0), (0, EP - Wg.shape[1])))
    out = _moe(xs, wg_pad, We, be)
    return out.reshape(x.shape)


# speedup vs baseline: 2.7721x; 2.7721x over previous
import functools

import jax
import jax.numpy as jnp
from jax.experimental import pallas as pl
from jax.experimental.pallas import tpu as pltpu

E = 8
K = 2
D = 768
EP = 128
BM = 256


def _moe_block(x_ref, wg_ref, we_ref, be_ref, o_ref):
    x_b = x_ref[...]
    logits = jnp.dot(x_b, wg_ref[...],
                     preferred_element_type=jnp.float32)
    lane = jax.lax.broadcasted_iota(jnp.int32, logits.shape, 1)
    logits = jnp.where(lane < E, logits, -1e30)

    v0 = jnp.max(logits, axis=1, keepdims=True)
    a0 = jnp.min(jnp.where(logits == v0, lane, EP), axis=1,
                 keepdims=True)
    logits2 = jnp.where(lane == a0, -1e30, logits)
    v1 = jnp.max(logits2, axis=1, keepdims=True)
    a1 = jnp.min(jnp.where(logits2 == v1, lane, EP), axis=1,
                 keepdims=True)

    w0 = 1.0 / (1.0 + jnp.exp(v1 - v0))
    w1 = 1.0 - w0

    acc = jnp.zeros((x_b.shape[0], D), dtype=jnp.float32)
    for e in range(E):
        w_e = jnp.where(a0 == e, w0, 0.0) + jnp.where(a1 == e, w1, 0.0)
        acc = acc + w_e * (jnp.dot(x_b, we_ref[e],
                                   preferred_element_type=jnp.float32)
                           + be_ref[e][None, :])
    o_ref[...] = acc


@jax.jit
def _moe(xs, wg_pad, We, be):
    T = xs.shape[0]
    grid = (T // BM,)
    return pl.pallas_call(
        _moe_block,
        grid=grid,
        in_specs=[
            pl.BlockSpec((BM, D), lambda i: (i, 0)),
            pl.BlockSpec((D, EP), lambda i: (0, 0)),
            pl.BlockSpec((E, D, D), lambda i: (0, 0, 0)),
            pl.BlockSpec((E, D), lambda i: (0, 0)),
        ],
        out_specs=pl.BlockSpec((BM, D), lambda i: (i, 0)),
        out_shape=jax.ShapeDtypeStruct((T, D), jnp.float32),
    )(xs, wg_pad, We, be)


def kernel(x, Wg, We, be):
    xs = x.reshape(-1, x.shape[-1])
    wg_pad = jnp.pad(Wg, ((0, 0), (0, EP - Wg.shape[1])))
    out = _moe(xs, wg_pad, We, be)
    return out.reshape(x.shape)


# speedup vs baseline: 2.9378x; 1.0598x over previous
import functools

import jax
import jax.numpy as jnp
from jax.experimental import pallas as pl
from jax.experimental.pallas import tpu as pltpu

E = 8
K = 2
D = 768
EP = 128
BM = 512


def _moe_block(x_ref, wg_ref, we_ref, be_ref, o_ref):
    x_b = x_ref[...]
    logits = jnp.dot(x_b, wg_ref[...],
                     preferred_element_type=jnp.float32)
    lane = jax.lax.broadcasted_iota(jnp.int32, logits.shape, 1)
    logits = jnp.where(lane < E, logits, -1e30)

    v0 = jnp.max(logits, axis=1, keepdims=True)
    a0 = jnp.min(jnp.where(logits == v0, lane, EP), axis=1,
                 keepdims=True)
    logits2 = jnp.where(lane == a0, -1e30, logits)
    v1 = jnp.max(logits2, axis=1, keepdims=True)
    a1 = jnp.min(jnp.where(logits2 == v1, lane, EP), axis=1,
                 keepdims=True)

    w0 = 1.0 / (1.0 + jnp.exp(v1 - v0))
    w1 = 1.0 - w0

    acc = jnp.zeros((x_b.shape[0], D), dtype=jnp.float32)
    for e in range(E):
        w_e = jnp.where(a0 == e, w0, 0.0) + jnp.where(a1 == e, w1, 0.0)
        acc = acc + w_e * (jnp.dot(x_b, we_ref[e],
                                   preferred_element_type=jnp.float32)
                           + be_ref[e][None, :])
    o_ref[...] = acc


@jax.jit
def _moe(xs, wg_pad, We, be):
    T = xs.shape[0]
    grid = (T // BM,)
    return pl.pallas_call(
        _moe_block,
        grid=grid,
        in_specs=[
            pl.BlockSpec((BM, D), lambda i: (i, 0)),
            pl.BlockSpec((D, EP), lambda i: (0, 0)),
            pl.BlockSpec((E, D, D), lambda i: (0, 0, 0)),
            pl.BlockSpec((E, D), lambda i: (0, 0)),
        ],
        out_specs=pl.BlockSpec((BM, D), lambda i: (i, 0)),
        out_shape=jax.ShapeDtypeStruct((T, D), jnp.float32),
    )(xs, wg_pad, We, be)


def kernel(x, Wg, We, be):
    xs = x.reshape(-1, x.shape[-1])
    wg_pad = jnp.pad(Wg, ((0, 0), (0, EP - Wg.shape[1])))
    out = _moe(xs, wg_pad, We, be)
    return out.reshape(x.shape)
